# baseline (speedup 1.0000x reference)
import functools

import jax
import jax.numpy as jnp
from jax import lax
from jax.experimental import pallas as pl
from jax.experimental.pallas import tpu as pltpu
from jax.experimental.pallas import tpu_sc as plsc

V = 60
NUM_WORKERS = 32
LANES = 16
CHUNK = 128
NSUB = 5
NTOK = 3
NWIN2 = V * V * V // LANES
NWIN3 = V * V * V * V // LANES


def _make_kernel(B, T):
    rows_w = B // NUM_WORKERS
    n_chunks = rows_w // CHUNK
    nsub = rows_w * NSUB
    nsub_chunks = nsub // CHUNK
    ntok = rows_w * NTOK
    ntok_chunks = ntok // CHUNK
    groups = rows_w // LANES
    mesh = plsc.VectorSubcoreMesh(core_axis_name="c", subcore_axis_name="s")

    @functools.partial(
        pl.kernel,
        out_type=jax.ShapeDtypeStruct((B, V), jnp.float32),
        mesh=mesh,
        compiler_params=pltpu.CompilerParams(
            needs_layout_passes=False, use_tc_tiling_on_sc=False),
        scratch_types=[
            pltpu.VMEM((rows_w,), jnp.int32),
            pltpu.VMEM((V + 1, V), jnp.float32),
            pltpu.VMEM((V,), jnp.int32),
            pltpu.VMEM((n_chunks, CHUNK), jnp.int32),
            pltpu.VMEM((n_chunks, CHUNK), jnp.int32),
            pltpu.VMEM((rows_w,), jnp.int32),
            pltpu.VMEM((rows_w,), jnp.int32),
            pltpu.VMEM((rows_w,), jnp.int32),
            pltpu.VMEM((nsub, LANES), jnp.float32),
            pltpu.VMEM((nsub, LANES), jnp.float32),
            pltpu.SemaphoreType.DMA,
            pltpu.SemaphoreType.DMA,
            pltpu.SemaphoreType.DMA,
        ],
    )
    def ngram_kernel(x_hbm, len_hbm, uni_hbm, t1_hbm, m1_hbm, m2_hbm, t2w_hbm,
                     t3w_hbm, m3_hbm, out_hbm, lenv, t01v, m1v, k3v,
                     m3v, codev, sv, s2v, r2w, r3w, sem3, sem2, semm3):
        cid = lax.axis_index("c")
        sid = lax.axis_index("s")
        wid = cid * 16 + sid
        base = wid * rows_w

        pltpu.sync_copy(len_hbm.at[pl.ds(base, rows_w)], lenv)
        pltpu.sync_copy(uni_hbm, t01v.at[0])
        pltpu.sync_copy(t1_hbm, t01v.at[pl.ds(1, V)])
        pltpu.sync_copy(m1_hbm, m1v)

        def stage12(m2v, pidx, tokv, ksv, ks2v):
            pltpu.sync_copy(m2_hbm, m2v)

            for g in range(groups):
                lens = lenv[pl.ds(g * LANES, LANES)]
                rows = lax.broadcasted_iota(jnp.int32, (LANES,), 0) + g * LANES
                rbase = (base + rows) * T
                for j in range(NTOK):
                    p = rbase + jnp.clip(lens - (j + 1), 0, T - 1)
                    ch = (j * rows_w + g * LANES) // CHUNK
                    off = (g * LANES) % CHUNK
                    pidx[ch, pl.ds(off, LANES)] = p

            tok_copies = [
                pltpu.async_copy(
                    x_hbm.at[pidx.at[ch]],
                    tokv.at[pl.ds(ch * CHUNK, CHUNK)], semm3)
                for ch in range(ntok_chunks)]
            for cp in tok_copies:
                cp.wait()

            for g in range(groups):
                lens = lenv[pl.ds(g * LANES, LANES)]
                rows = lax.broadcasted_iota(jnp.int32, (LANES,), 0) + g * LANES
                a = tokv[pl.ds(g * LANES, LANES)]
                b = tokv[pl.ds(rows_w + g * LANES, LANES)]
                c = tokv[pl.ds(2 * rows_w + g * LANES, LANES)]
                key2 = b * V + a
                key3 = c * (V * V) + key2
                m1bit = plsc.load_gather(m1v, [a])
                m2bit = plsc.load_gather(m2v, [key2])
                take2 = jnp.logical_and(lens >= 2, m2bit != 0)
                take1 = jnp.logical_and(lens >= 1, m1bit != 0)
                src01 = jnp.where(take1, a + 1, 0)
                ch = g * LANES // CHUNK
                off = (g * LANES) % CHUNK
                k3v[ch, pl.ds(off, LANES)] = key3
                q3 = (key3 * 15) >> 2
                q2 = (key2 * 15) >> 2
                sv[pl.ds(g * LANES, LANES)] = key3 * 60 - q3 * 16
                s2v[pl.ds(g * LANES, LANES)] = key2 * 60 - q2 * 16
                pbase = rows * NSUB
                for j in range(NSUB):
                    plsc.store_scatter(ksv, [pbase + j],
                                       jnp.minimum(q3 + j, NWIN3 - 1))
                    plsc.store_scatter(ks2v, [pbase + j],
                                       jnp.minimum(q2 + j, NWIN2 - 1))
                codev[pl.ds(g * LANES, LANES)] = jnp.where(take2, 1, 2 + src01)

            copies = []
            for ch in range(nsub_chunks):
                copies.append(pltpu.async_copy(
                    t3w_hbm.at[ksv.at[pl.ds(ch * CHUNK, CHUNK)]],
                    r3w.at[pl.ds(ch * CHUNK, CHUNK)], sem3))
                copies.append(pltpu.async_copy(
                    t2w_hbm.at[ks2v.at[pl.ds(ch * CHUNK, CHUNK)]],
                    r2w.at[pl.ds(ch * CHUNK, CHUNK)], sem2))
            for ch in range(n_chunks):
                copies.append(pltpu.async_copy(
                    m3_hbm.at[k3v.at[ch]], m3v.at[ch], semm3))
            for cp in copies:
                cp.wait()

        pl.run_scoped(stage12,
                      pltpu.VMEM((V * V,), jnp.int32),
                      pltpu.VMEM((ntok_chunks, CHUNK), jnp.int32),
                      pltpu.VMEM((ntok,), jnp.int32),
                      pltpu.VMEM((nsub,), jnp.int32),
                      pltpu.VMEM((nsub,), jnp.int32))

        def stage3(outv):
            for g in range(groups):
                ch = g * LANES // CHUNK
                off = (g * LANES) % CHUNK
                lens = lenv[pl.ds(g * LANES, LANES)]
                m3bit = m3v[ch, pl.ds(off, LANES)]
                take3 = jnp.logical_and(lens >= 3, m3bit != 0)
                old = codev[pl.ds(g * LANES, LANES)]
                codev[pl.ds(g * LANES, LANES)] = jnp.where(take3, 0, old)

            lane = lax.broadcasted_iota(jnp.int32, (LANES,), 0)

            def g_body(g, carry):
                row0 = g * LANES
                codes = codev[pl.ds(row0, LANES)]
                svec = sv[pl.ds(row0, LANES)]
                s2vec = s2v[pl.ds(row0, LANES)]
                for l in range(LANES):
                    c = codes[l]
                    r = row0 + l

                    @pl.when(c <= 1)
                    def _():
                        use3 = c == 0
                        pos0 = r * (NSUB * LANES) + lane + jnp.where(
                            use3, svec[l], s2vec[l])
                        for o in (0, 16, 32, V - LANES):
                            pos = pos0 + o
                            v3 = plsc.load_gather(r3w, [pos >> 4, pos & 15])
                            v2 = plsc.load_gather(r2w, [pos >> 4, pos & 15])
                            outv[r, pl.ds(o, LANES)] = jnp.where(use3, v3, v2)

                    @pl.when(c >= 2)
                    def _():
                        src = c - 2
                        for o in (0, 16, 32, V - LANES):
                            outv[r, pl.ds(o, LANES)] = t01v[
                                src, pl.ds(o, LANES)]

                return carry

            lax.fori_loop(0, groups, g_body, 0)

            pltpu.sync_copy(outv, out_hbm.at[pl.ds(base, rows_w)])

        pl.run_scoped(stage3, pltpu.VMEM((rows_w, V), jnp.float32))

    return ngram_kernel


@jax.jit
def kernel(x, lengths, uni, table1, table2, table3, mask1, mask2, mask3):
    B, T = x.shape
    m1 = mask1.astype(jnp.int32)
    m2 = mask2.astype(jnp.int32)
    m3 = mask3.astype(jnp.int32)
    out = _make_kernel(B, T)(
        x.reshape(-1), lengths.astype(jnp.int32), uni, table1,
        m1, m2, table2.reshape(NWIN2, LANES),
        table3.reshape(NWIN3, LANES), m3)
    return out[:, None, :]
